# baseline (speedup 1.0000x reference)
import functools

import jax
import jax.numpy as jnp
from jax import lax
from jax.experimental import pallas as pl
from jax.experimental.pallas import tpu as pltpu
from jax.experimental.pallas import tpu_sc as plsc

VOCAB = 1000000
SUFF_PAD = 1024
N_TOKENS = 16384
HALF_DIM = 64
NBUF = 8

_info = plsc.get_sparse_core_info()
NC, NS = _info.num_cores, _info.num_subcores
NW = NC * NS
B_PER_W = N_TOKENS // NW
PASS = 128
N_PASS = B_PER_W // PASS


def _make_kernel():
    mesh = plsc.VectorSubcoreMesh(core_axis_name="c", subcore_axis_name="s")

    @functools.partial(
        pl.kernel,
        mesh=mesh,
        out_type=jax.ShapeDtypeStruct((N_TOKENS, 2 * HALF_DIM), jnp.float32),
        scratch_types=[
            pltpu.VMEM((B_PER_W,), jnp.int32),
            pltpu.VMEM((B_PER_W,), jnp.int32),
            pltpu.VMEM((B_PER_W,), jnp.int32),
            pltpu.VMEM((4, 8, SUFF_PAD // 128, 128), jnp.int32),
            [pltpu.VMEM((8, 8, 128), jnp.float32) for _ in range(NBUF)],
            pltpu.VMEM((PASS, 2 * HALF_DIM), jnp.float32),
            [pltpu.SemaphoreType.DMA for _ in range(NBUF)],
            pltpu.SemaphoreType.DMA,
        ],
        compiler_params=pltpu.CompilerParams(needs_layout_passes=False),
    )
    def k(word_idx_hbm, suff_idx_hbm, w3_hbm, s4_hbm, out_hbm,
          idx_w, idx_s, combo_v, suff_v, blks, outv, sems, ssem):
        wid = lax.axis_index("s") * NC + lax.axis_index("c")
        base = wid * B_PER_W
        scopy = pltpu.make_async_copy(s4_hbm, suff_v, ssem)
        scopy.start()
        pltpu.sync_copy(word_idx_hbm.at[pl.ds(base, B_PER_W)], idx_w)
        pltpu.sync_copy(suff_idx_hbm.at[pl.ds(base, B_PER_W)], idx_s)

        def pack_body(g, _):
            cw = idx_w[pl.ds(g * 16, 16)]
            cs = idx_s[pl.ds(g * 16, 16)]
            combo_v[pl.ds(g * 16, 16)] = (
                lax.shift_left(lax.shift_right_logical(cw, 7), 17)
                | lax.shift_left(cw & 127, 10)
                | (cs << 0))
            return 0

        lax.fori_loop(0, B_PER_W // 16, pack_body, 0)
        scopy.wait()

        iota = lax.iota(jnp.int32, 16)
        lane_hi = iota >> 3
        lane_lo = iota & 7
        pair_lo = iota >> 1
        parity = iota & 1
        lo_mask = jnp.full((16,), 0xFFFF, jnp.int32)
        sixteen = jnp.full((16,), 16, jnp.int32)

        def scalar_at(t):
            grp = (t >> 4) << 4
            lane = t & 15
            v = combo_v[pl.ds(grp, 16)]
            sel = jnp.where(iota == jnp.full((16,), lane, jnp.int32), v,
                            jnp.zeros((16,), jnp.int32))
            return jnp.sum(sel)

        def start_word(t, u):
            b = scalar_at(t) >> 17
            pltpu.make_async_copy(
                w3_hbm.at[:, :, pl.ds(pl.multiple_of(b * 128, 128), 128)],
                blks[u], sems[u]).start()

        def wait_blk(u):
            pltpu.make_async_copy(
                w3_hbm.at[:, :, pl.ds(0, 128)], blks[u], sems[u]).wait()

        def extract(t, row, u):
            c = scalar_at(t)
            wl = (c >> 10) & 127
            si = c & 1023
            sj = si >> 7
            sl = si & 127
            wl_v = jnp.full((16,), wl, jnp.int32)
            sj_v = jnp.full((16,), sj, jnp.int32)
            sl_v = jnp.full((16,), sl, jnp.int32)
            for cb in range(4):
                d0 = lane_hi + (2 * cb)
                wv = plsc.load_gather(blks[u], [d0, lane_lo, wl_v])
                outv[row, pl.ds(cb * 16, 16)] = wv
                pw = plsc.load_gather(
                    suff_v, [jnp.full((16,), cb, jnp.int32), pair_lo, sj_v, sl_v])
                half = jnp.where(parity == 1,
                                 lax.shift_right_logical(pw, sixteen),
                                 pw & lo_mask)
                sv = plsc.bitcast(lax.shift_left(half, sixteen), jnp.float32)
                outv[row, pl.ds(HALF_DIM + cb * 16, 16)] = sv

        for h in range(N_PASS):
            pbase = h * PASS
            for u in range(NBUF):
                start_word(pbase + u, u)

            def body(i, _):
                for u in range(NBUF):
                    t = pbase + NBUF * i + u
                    wait_blk(u)
                    extract(t, NBUF * i + u, u)

                    @pl.when(i < PASS // NBUF - 1)
                    def _():
                        start_word(t + NBUF, u)

                return 0

            lax.fori_loop(0, PASS // NBUF, body, 0)
            pltpu.sync_copy(outv, out_hbm.at[pl.ds(base + pbase, PASS)])

    return k


_sc_lookup = _make_kernel()


def kernel(word_idx, suff_idx, W_word, W_suff):
    w3 = W_word.T.reshape(8, 8, VOCAB)
    sp = jnp.pad(W_suff, ((0, SUFF_PAD - W_suff.shape[0]), (0, 0)))
    spb = sp.astype(jnp.bfloat16).reshape(SUFF_PAD, HALF_DIM // 2, 2)
    packed = jax.lax.bitcast_convert_type(spb, jnp.int32)
    s4 = packed.T.reshape(4, 8, SUFF_PAD // 128, 128)
    return _sc_lookup(word_idx.astype(jnp.int32), suff_idx.astype(jnp.int32),
                      w3, s4)


# speedup vs baseline: 1.0079x; 1.0079x over previous
import functools

import jax
import jax.numpy as jnp
from jax import lax
from jax.experimental import pallas as pl
from jax.experimental.pallas import tpu as pltpu
from jax.experimental.pallas import tpu_sc as plsc

VOCAB = 1000000
SUFF_PAD = 1024
N_TOKENS = 16384
HALF_DIM = 64
NBUF = 8

_info = plsc.get_sparse_core_info()
NC, NS = _info.num_cores, _info.num_subcores
NW = NC * NS
B_PER_W = N_TOKENS // NW
PASS = 128
N_PASS = B_PER_W // PASS


def _make_kernel():
    mesh = plsc.VectorSubcoreMesh(core_axis_name="c", subcore_axis_name="s")

    @functools.partial(
        pl.kernel,
        mesh=mesh,
        out_type=jax.ShapeDtypeStruct((N_TOKENS, 2 * HALF_DIM), jnp.float32),
        scratch_types=[
            pltpu.VMEM((B_PER_W,), jnp.int32),
            pltpu.VMEM((B_PER_W,), jnp.int32),
            pltpu.VMEM((B_PER_W,), jnp.int32),
            pltpu.VMEM((B_PER_W,), jnp.int32),
            pltpu.VMEM((B_PER_W,), jnp.int32),
            pltpu.VMEM((B_PER_W,), jnp.int32),
            pltpu.VMEM((4, 8, SUFF_PAD // 128, 128), jnp.int32),
            [pltpu.VMEM((8, 8, 128), jnp.float32) for _ in range(NBUF)],
            pltpu.VMEM((PASS, 2 * HALF_DIM), jnp.float32),
            [pltpu.SemaphoreType.DMA for _ in range(NBUF)],
            pltpu.SemaphoreType.DMA,
        ],
        compiler_params=pltpu.CompilerParams(needs_layout_passes=False),
    )
    def k(word_idx_hbm, suff_idx_hbm, w3_hbm, s4_hbm, out_hbm,
          idx_w, idx_s, b_arr, wl_arr, sj_arr, sl_arr, suff_v, blks, outv, sems, ssem):
        wid = lax.axis_index("s") * NC + lax.axis_index("c")
        base = wid * B_PER_W
        scopy = pltpu.make_async_copy(s4_hbm, suff_v, ssem)
        scopy.start()
        pltpu.sync_copy(word_idx_hbm.at[pl.ds(base, B_PER_W)], idx_w)
        pltpu.sync_copy(suff_idx_hbm.at[pl.ds(base, B_PER_W)], idx_s)

        def pack_body(g, _):
            cw = idx_w[pl.ds(g * 16, 16)]
            cs = idx_s[pl.ds(g * 16, 16)]
            b_arr[pl.ds(g * 16, 16)] = lax.shift_right_logical(cw, 7)
            wl_arr[pl.ds(g * 16, 16)] = cw & 127
            sj_arr[pl.ds(g * 16, 16)] = lax.shift_right_logical(cs, 7)
            sl_arr[pl.ds(g * 16, 16)] = cs & 127
            return 0

        lax.fori_loop(0, B_PER_W // 16, pack_body, 0)
        scopy.wait()

        iota = lax.iota(jnp.int32, 16)
        lane_hi = iota >> 3
        lane_lo = iota & 7
        pair_lo = iota >> 1
        parity = iota & 1
        lo_mask = jnp.full((16,), 0xFFFF, jnp.int32)
        sixteen = jnp.full((16,), 16, jnp.int32)

        def scalar_at(t):
            grp = (t >> 4) << 4
            lane = t & 15
            v = b_arr[pl.ds(grp, 16)]
            sel = jnp.where(iota == jnp.full((16,), lane, jnp.int32), v,
                            jnp.zeros((16,), jnp.int32))
            return jnp.sum(sel)

        def start_word(t, u):
            b = scalar_at(t)
            pltpu.make_async_copy(
                w3_hbm.at[:, :, pl.ds(pl.multiple_of(b * 128, 128), 128)],
                blks[u], sems[u]).start()

        def wait_blk(u):
            pltpu.make_async_copy(
                w3_hbm.at[:, :, pl.ds(0, 128)], blks[u], sems[u]).wait()

        def extract(t, row, u):
            t_v = jnp.full((16,), t, jnp.int32)
            wl_v = plsc.load_gather(wl_arr, [t_v])
            sj_v = plsc.load_gather(sj_arr, [t_v])
            sl_v = plsc.load_gather(sl_arr, [t_v])
            for cb in range(4):
                d0 = lane_hi + (2 * cb)
                wv = plsc.load_gather(blks[u], [d0, lane_lo, wl_v])
                outv[row, pl.ds(cb * 16, 16)] = wv
                pw = plsc.load_gather(
                    suff_v, [jnp.full((16,), cb, jnp.int32), pair_lo, sj_v, sl_v])
                half = jnp.where(parity == 1,
                                 lax.shift_right_logical(pw, sixteen),
                                 pw & lo_mask)
                sv = plsc.bitcast(lax.shift_left(half, sixteen), jnp.float32)
                outv[row, pl.ds(HALF_DIM + cb * 16, 16)] = sv

        for h in range(N_PASS):
            pbase = h * PASS
            for u in range(NBUF):
                start_word(pbase + u, u)

            def body(i, _):
                for u in range(NBUF):
                    t = pbase + NBUF * i + u
                    wait_blk(u)
                    extract(t, NBUF * i + u, u)

                    @pl.when(i < PASS // NBUF - 1)
                    def _():
                        start_word(t + NBUF, u)

                return 0

            lax.fori_loop(0, PASS // NBUF, body, 0)
            pltpu.sync_copy(outv, out_hbm.at[pl.ds(base + pbase, PASS)])

    return k


_sc_lookup = _make_kernel()


def kernel(word_idx, suff_idx, W_word, W_suff):
    w3 = W_word.T.reshape(8, 8, VOCAB)
    sp = jnp.pad(W_suff, ((0, SUFF_PAD - W_suff.shape[0]), (0, 0)))
    spb = sp.astype(jnp.bfloat16).reshape(SUFF_PAD, HALF_DIM // 2, 2)
    packed = jax.lax.bitcast_convert_type(spb, jnp.int32)
    s4 = packed.T.reshape(4, 8, SUFF_PAD // 128, 128)
    return _sc_lookup(word_idx.astype(jnp.int32), suff_idx.astype(jnp.int32),
                      w3, s4)
